# Initial kernel scaffold; baseline (speedup 1.0000x reference)
import jax
import jax.numpy as jnp
from jax.experimental import pallas as pl


def kernel(x, edge_index, edge_attr, Wn1, bn1, Wn2, bn2, We1, be1, We2, be2, Wl, Wr, Wedge, att, bias):
    raise NotImplementedError("write your pallas kernel here")


# speedup vs baseline: 7.4454x; 7.4454x over previous
import functools

import jax
import jax.numpy as jnp
from jax import lax
from jax.experimental import pallas as pl
from jax.experimental.pallas import tpu as pltpu
from jax.experimental.pallas import tpu_sc as plsc

N_NODES = 10000
N_EDGES = 320000
OUT = 8

NUM_TILES = 32
EPT = N_EDGES // NUM_TILES
CHUNK = 400
NCHUNK = EPT // CHUNK
SUB = 80
NSUB = CHUNK // SUB
GROUPS = CHUNK // 16

_P = jax.lax.Precision.HIGHEST


def _dot(a, b):
    return jnp.dot(a, b, precision=_P, preferred_element_type=jnp.float32)


def _node_body(x_ref, wn1_ref, bn1_ref, wn2_ref, bn2_ref, wl_ref, wr_ref,
               t_ref):
    h1 = jnp.maximum(_dot(x_ref[...], wn1_ref[...]) + bn1_ref[...][None, :],
                     0.0)
    wlr = jnp.concatenate([wl_ref[...], wr_ref[...]], axis=1)
    w2 = _dot(wn2_ref[...], wlr)
    b2 = _dot(bn2_ref[...][None, :], wlr)
    t_ref[...] = _dot(h1, w2) + b2


def _node_encode(x, wn1, bn1, wn2, bn2, wl, wr):
    return pl.pallas_call(
        _node_body,
        out_shape=jax.ShapeDtypeStruct((N_NODES, 16), jnp.float32),
    )(x, wn1, bn1, wn2, bn2, wl, wr)


_EBLK = 8000
_EGRID = N_EDGES // _EBLK


def _edge_body(ea_ref, we1_ref, be1_ref, we2_ref, wedge_ref, be2_ref, ee_ref):
    a = jnp.maximum(_dot(ea_ref[...], we1_ref[...]) + be1_ref[...][None, :],
                    0.0)
    w2e = _dot(we2_ref[...], wedge_ref[...])
    b2e = _dot(be2_ref[...][None, :], wedge_ref[...])
    ee_ref[...] = _dot(a, w2e) + b2e


def _edge_encode(edge_attr, we1, be1, we2, wedge, be2):
    return pl.pallas_call(
        _edge_body,
        grid=(_EGRID,),
        in_specs=[
            pl.BlockSpec((_EBLK, 16), lambda i: (i, 0)),
            pl.BlockSpec((16, 128), lambda i: (0, 0)),
            pl.BlockSpec((128,), lambda i: (0,)),
            pl.BlockSpec((128, 128), lambda i: (0, 0)),
            pl.BlockSpec((128, 8), lambda i: (0, 0)),
            pl.BlockSpec((128,), lambda i: (0,)),
        ],
        out_specs=pl.BlockSpec((_EBLK, 8), lambda i: (i, 0)),
        out_shape=jax.ShapeDtypeStruct((N_EDGES, 8), jnp.float32),
    )(edge_attr, we1, be1, we2, wedge, be2)


def _sc_attend(t_tab, src, dst, ee, att16):
    mesh = plsc.VectorSubcoreMesh(core_axis_name="c", subcore_axis_name="s")

    @functools.partial(
        pl.kernel,
        out_type=(
            jax.ShapeDtypeStruct((NUM_TILES * N_NODES * OUT,), jnp.float32),
            jax.ShapeDtypeStruct((NUM_TILES * N_NODES,), jnp.float32),
        ),
        mesh=mesh,
        compiler_params=pltpu.CompilerParams(use_tc_tiling_on_sc=False,
                                             needs_layout_passes=False),
        scratch_types=[
            pltpu.VMEM((N_NODES * OUT,), jnp.float32),
            pltpu.VMEM((N_NODES,), jnp.float32),
            pltpu.VMEM((2, CHUNK, 16), jnp.float32),
            pltpu.VMEM((2, CHUNK, 16), jnp.float32),
            pltpu.VMEM((2, CHUNK, 8), jnp.float32),
            pltpu.VMEM((3, CHUNK), jnp.int32),
            pltpu.VMEM((3, CHUNK), jnp.int32),
            pltpu.VMEM((OUT, 16), jnp.float32),
            pltpu.SemaphoreType.DMA,
            pltpu.SemaphoreType.DMA,
            pltpu.SemaphoreType.DMA,
        ],
    )
    def body(t_hbm, src_hbm, dst_hbm, ee_hbm, att_hbm, nums_hbm, dens_hbm,
             num_v, den_v, srow_v, drow_v, ee_v, sidx_v, didx_v, att_v,
             sem_i, sem_r0, sem_r1):
        wid = lax.axis_index("s") * 2 + lax.axis_index("c")
        sem_r = [sem_r0, sem_r1]

        zero = jnp.zeros((16,), jnp.float32)

        def _znum(i, carry):
            base = i * 400
            for u in range(25):
                num_v[pl.ds(base + u * 16, 16)] = zero
            return carry

        def _zden(i, carry):
            base = i * 400
            for u in range(25):
                den_v[pl.ds(base + u * 16, 16)] = zero
            return carry

        lax.fori_loop(0, (N_NODES * OUT) // 400, _znum, 0)
        lax.fori_loop(0, N_NODES // 400, _zden, 0)

        pltpu.sync_copy(att_hbm, att_v)
        att_b = [att_v[f, :] for f in range(OUT)]

        def issue_idx(c):
            b3 = c % 3
            sl = pl.ds(wid * EPT + c * CHUNK, CHUNK)
            return [
                pltpu.async_copy(src_hbm.at[sl], sidx_v.at[b3], sem_i),
                pltpu.async_copy(dst_hbm.at[sl], didx_v.at[b3], sem_i),
            ]

        def issue_rows(c):
            b2, b3 = c % 2, c % 3
            hs = []
            for j in range(NSUB):
                sl = pl.ds(j * SUB, SUB)
                hs.append(pltpu.async_copy(
                    t_hbm.at[sidx_v.at[b3, sl]], srow_v.at[b2, sl],
                    sem_r[b2]))
                hs.append(pltpu.async_copy(
                    t_hbm.at[didx_v.at[b3, sl]], drow_v.at[b2, sl],
                    sem_r[b2]))
            hs.append(pltpu.async_copy(
                ee_hbm.at[pl.ds(wid * EPT + c * CHUNK, CHUNK)],
                ee_v.at[b2], sem_r[b2]))
            return hs

        def compute_chunk(c):
            b2, b3 = c % 2, c % 3
            srow = srow_v.at[b2]
            drow = drow_v.at[b2]
            eer = ee_v.at[b2]

            def gbody(g, carry):
                row = g * 16 + lax.iota(jnp.int32, 16)
                dstv = didx_v[b3, pl.ds(g * 16, 16)]
                acc = jnp.zeros((16,), jnp.float32)
                sls = []
                for f in range(OUT):
                    cf = jnp.full((16,), f, jnp.int32)
                    sl_f = plsc.load_gather(srow, [row, cf])
                    sr_f = plsc.load_gather(
                        drow, [row, jnp.full((16,), 8 + f, jnp.int32)])
                    e_f = plsc.load_gather(eer, [row, cf])
                    m = sl_f + sr_f + e_f
                    lr = jnp.maximum(m, 0.2 * m)
                    acc = acc + att_b[f] * lr
                    sls.append(sl_f)
                ex = jnp.exp(acc)
                plsc.addupdate_scatter(den_v, [dstv], ex)
                for f in range(OUT):
                    plsc.addupdate_scatter(num_v, [dstv + f * N_NODES],
                                           ex * sls[f])
                return carry

            lax.fori_loop(0, GROUPS, gbody, 0)

        for h in issue_idx(0):
            h.wait()
        rows_pend = {0: issue_rows(0)}
        idx_pend = {1: issue_idx(1)}
        for c in range(NCHUNK):
            if c + 1 < NCHUNK:
                for h in idx_pend.pop(c + 1):
                    h.wait()
                rows_pend[c + 1] = issue_rows(c + 1)
            if c + 2 < NCHUNK:
                idx_pend[c + 2] = issue_idx(c + 2)
            for h in rows_pend.pop(c):
                h.wait()
            compute_chunk(c)

        pltpu.sync_copy(num_v, nums_hbm.at[pl.ds(wid * N_NODES * OUT,
                                                 N_NODES * OUT)])
        pltpu.sync_copy(den_v, dens_hbm.at[pl.ds(wid * N_NODES, N_NODES)])

    return body(t_tab, src, dst, ee, att16)


def _combine_body(nums_ref, dens_ref, bias_ref, out_ref):
    num = jnp.sum(nums_ref[...], axis=0)
    den = jnp.sum(dens_ref[...], axis=0)
    out_ref[...] = (num / (den[None, :] + 1e-16)
                    + bias_ref[...][:, None])


def _combine(nums, dens, bias):
    return pl.pallas_call(
        _combine_body,
        out_shape=jax.ShapeDtypeStruct((OUT, N_NODES), jnp.float32),
    )(nums, dens, bias)


def kernel(x, edge_index, edge_attr, Wn1, bn1, Wn2, bn2, We1, be1, We2, be2,
           Wl, Wr, Wedge, att, bias):
    src = edge_index[0]
    dst = edge_index[1]
    att_tab = jnp.tile(att[:, None], (1, 16))

    t_tab = _node_encode(x, Wn1, bn1, Wn2, bn2, Wl, Wr)
    ee = _edge_encode(edge_attr, We1, be1, We2, Wedge, be2)
    nums, dens = _sc_attend(t_tab, src, dst, ee, att_tab)
    out_t = _combine(nums.reshape(NUM_TILES, OUT, N_NODES),
                     dens.reshape(NUM_TILES, N_NODES), bias)
    return out_t.T


# speedup vs baseline: 16.9337x; 2.2744x over previous
import functools

import jax
import jax.numpy as jnp
from jax import lax
from jax.experimental import pallas as pl
from jax.experimental.pallas import tpu as pltpu
from jax.experimental.pallas import tpu_sc as plsc

N_NODES = 10000
N_PAD = 10240
N_EDGES = 320000
OUT = 8

NUM_TILES = 32
EPT = N_EDGES // NUM_TILES
CHUNK = 400
NCHUNK = EPT // CHUNK
SUB = 80
NSUB = CHUNK // SUB
GROUPS = CHUNK // 16

_P = jax.lax.Precision.HIGHEST


def _dot(a, b):
    return jnp.dot(a, b, precision=_P, preferred_element_type=jnp.float32)


def _node_body(x_ref, wn1_ref, bn1_ref, wn2_ref, bn2_ref, wl_ref, wr_ref,
               t_ref):
    h1 = jnp.maximum(_dot(x_ref[...], wn1_ref[...]) + bn1_ref[...][None, :],
                     0.0)
    wlr = jnp.concatenate([wl_ref[...], wr_ref[...]], axis=1)
    w2 = _dot(wn2_ref[...], wlr)
    b2 = _dot(bn2_ref[...][None, :], wlr)
    t_ref[...] = _dot(h1, w2) + b2


def _node_encode(x, wn1, bn1, wn2, bn2, wl, wr):
    return pl.pallas_call(
        _node_body,
        out_shape=jax.ShapeDtypeStruct((N_NODES, 16), jnp.float32),
    )(x, wn1, bn1, wn2, bn2, wl, wr)


_EBLK = 6400
_EGRID = N_EDGES // _EBLK


def _edge_body(eat_ref, we1t_ref, be1_ref, we2t_ref, wedget_ref, be2_ref,
               *out_refs):
    i = pl.program_id(0)
    h = jnp.maximum(_dot(we1t_ref[...], eat_ref[...])
                    + be1_ref[...][:, None], 0.0)
    w2et = _dot(wedget_ref[...], we2t_ref[...])
    b2et = _dot(wedget_ref[...], be2_ref[...][:, None])
    eet = _dot(w2et, h) + b2et
    base = pl.multiple_of(i * _EBLK, 128)
    for f in range(OUT):
        out_refs[f][pl.ds(base, _EBLK)] = eet[f, :]


def _edge_encode(eat, we1t, be1, we2t, wedget, be2):
    return pl.pallas_call(
        _edge_body,
        grid=(_EGRID,),
        in_specs=[
            pl.BlockSpec((16, _EBLK), lambda i: (0, i)),
            pl.BlockSpec((128, 16), lambda i: (0, 0)),
            pl.BlockSpec((128,), lambda i: (0,)),
            pl.BlockSpec((128, 128), lambda i: (0, 0)),
            pl.BlockSpec((8, 128), lambda i: (0, 0)),
            pl.BlockSpec((128,), lambda i: (0,)),
        ],
        out_specs=[pl.BlockSpec((N_EDGES,), lambda i: (0,))
                   for _ in range(OUT)],
        out_shape=[jax.ShapeDtypeStruct((N_EDGES,), jnp.float32)
                   for _ in range(OUT)],
    )(eat, we1t, be1, we2t, wedget, be2)


def _sc_attend(t_tab, src, dst, ee_list, att_tab):
    mesh = plsc.VectorSubcoreMesh(core_axis_name="c", subcore_axis_name="s")

    @functools.partial(
        pl.kernel,
        out_type=(
            jax.ShapeDtypeStruct((NUM_TILES * N_PAD * OUT,), jnp.float32),
            jax.ShapeDtypeStruct((NUM_TILES * N_PAD,), jnp.float32),
        ),
        mesh=mesh,
        compiler_params=pltpu.CompilerParams(use_tc_tiling_on_sc=False,
                                             needs_layout_passes=False),
        scratch_types=[
            pltpu.VMEM((N_PAD * OUT,), jnp.float32),
            pltpu.VMEM((N_PAD,), jnp.float32),
            pltpu.VMEM((2, CHUNK, 16), jnp.float32),
            pltpu.VMEM((2, CHUNK, 16), jnp.float32),
            pltpu.VMEM((2, OUT, CHUNK), jnp.float32),
            pltpu.VMEM((3, CHUNK), jnp.int32),
            pltpu.VMEM((3, CHUNK), jnp.int32),
            pltpu.VMEM((OUT, 16), jnp.float32),
            pltpu.SemaphoreType.DMA,
            pltpu.SemaphoreType.DMA,
            pltpu.SemaphoreType.DMA,
        ],
    )
    def body(t_hbm, src_hbm, dst_hbm, e0, e1, e2, e3, e4, e5, e6, e7,
             att_hbm, nums_hbm, dens_hbm,
             num_v, den_v, srow_v, drow_v, ee_v, sidx_v, didx_v, att_v,
             sem_i, sem_r0, sem_r1):
        wid = lax.axis_index("s") * 2 + lax.axis_index("c")
        sem_r = [sem_r0, sem_r1]
        ee_hbms = [e0, e1, e2, e3, e4, e5, e6, e7]

        zero = jnp.zeros((16,), jnp.float32)

        def _znum(i, carry):
            base = i * 256
            for u in range(16):
                num_v[pl.ds(base + u * 16, 16)] = zero
            return carry

        def _zden(i, carry):
            base = i * 256
            for u in range(16):
                den_v[pl.ds(base + u * 16, 16)] = zero
            return carry

        lax.fori_loop(0, (N_PAD * OUT) // 256, _znum, 0)
        lax.fori_loop(0, N_PAD // 256, _zden, 0)

        pltpu.sync_copy(att_hbm, att_v)
        att_b = [att_v[f, :] for f in range(OUT)]

        def issue_idx(c):
            b3 = c % 3
            sl = pl.ds(wid * EPT + c * CHUNK, CHUNK)
            return [
                pltpu.async_copy(src_hbm.at[sl], sidx_v.at[b3], sem_i),
                pltpu.async_copy(dst_hbm.at[sl], didx_v.at[b3], sem_i),
            ]

        def issue_rows(c):
            b2, b3 = c % 2, c % 3
            hs = []
            for j in range(NSUB):
                sl = pl.ds(j * SUB, SUB)
                hs.append(pltpu.async_copy(
                    t_hbm.at[sidx_v.at[b3, sl]], srow_v.at[b2, sl],
                    sem_r[b2]))
                hs.append(pltpu.async_copy(
                    t_hbm.at[didx_v.at[b3, sl]], drow_v.at[b2, sl],
                    sem_r[b2]))
            esl = pl.ds(wid * EPT + c * CHUNK, CHUNK)
            for f in range(OUT):
                hs.append(pltpu.async_copy(
                    ee_hbms[f].at[esl], ee_v.at[b2, f], sem_r[b2]))
            return hs

        def compute_chunk(c):
            b2, b3 = c % 2, c % 3
            srow = srow_v.at[b2]
            drow = drow_v.at[b2]

            def gbody(g, carry):
                row = g * 16 + lax.iota(jnp.int32, 16)
                dstv = didx_v[b3, pl.ds(g * 16, 16)]
                acc = jnp.zeros((16,), jnp.float32)
                sls = []
                for f in range(OUT):
                    cf = jnp.full((16,), f, jnp.int32)
                    sl_f = plsc.load_gather(srow, [row, cf])
                    sr_f = plsc.load_gather(
                        drow, [row, jnp.full((16,), 8 + f, jnp.int32)])
                    e_f = ee_v[b2, f, pl.ds(g * 16, 16)]
                    m = sl_f + sr_f + e_f
                    lr = jnp.maximum(m, 0.2 * m)
                    acc = acc + att_b[f] * lr
                    sls.append(sl_f)
                ex = jnp.exp(acc)
                plsc.addupdate_scatter(den_v, [dstv], ex)
                for f in range(OUT):
                    plsc.addupdate_scatter(num_v, [dstv + f * N_PAD],
                                           ex * sls[f])
                return carry

            lax.fori_loop(0, GROUPS, gbody, 0)

        for h in issue_idx(0):
            h.wait()
        rows_pend = {0: issue_rows(0)}
        idx_pend = {1: issue_idx(1)}
        for c in range(NCHUNK):
            if c + 1 < NCHUNK:
                for h in idx_pend.pop(c + 1):
                    h.wait()
                rows_pend[c + 1] = issue_rows(c + 1)
            if c + 2 < NCHUNK:
                idx_pend[c + 2] = issue_idx(c + 2)
            for h in rows_pend.pop(c):
                h.wait()
            compute_chunk(c)

        pltpu.sync_copy(num_v, nums_hbm.at[pl.ds(wid * N_PAD * OUT,
                                                 N_PAD * OUT)])
        pltpu.sync_copy(den_v, dens_hbm.at[pl.ds(wid * N_PAD, N_PAD)])

    return body(t_tab, src, dst, *ee_list, att_tab)


def _combine_body(nums_ref, dens_ref, biasr_ref, out_ref, dacc_ref):
    i = pl.program_id(0)

    @pl.when(i == 0)
    def _init():
        out_ref[...] = jnp.zeros_like(out_ref)
        dacc_ref[...] = jnp.zeros_like(dacc_ref)

    out_ref[...] = out_ref[...] + nums_ref[...]
    dacc_ref[...] = dacc_ref[...] + dens_ref[...]

    @pl.when(i == NUM_TILES - 1)
    def _final():
        den_rep = jnp.concatenate([dacc_ref[...]] * OUT)
        out_ref[...] = out_ref[...] / (den_rep + 1e-16) + biasr_ref[...]


def _combine(nums, dens, bias_rep):
    return pl.pallas_call(
        _combine_body,
        grid=(NUM_TILES,),
        in_specs=[
            pl.BlockSpec((N_PAD * OUT,), lambda i: (i,)),
            pl.BlockSpec((N_PAD,), lambda i: (i,)),
            pl.BlockSpec((N_PAD * OUT,), lambda i: (0,)),
        ],
        out_specs=pl.BlockSpec((N_PAD * OUT,), lambda i: (0,)),
        out_shape=jax.ShapeDtypeStruct((N_PAD * OUT,), jnp.float32),
        scratch_shapes=[pltpu.VMEM((N_PAD,), jnp.float32)],
    )(nums, dens, bias_rep)


def kernel(x, edge_index, edge_attr, Wn1, bn1, Wn2, bn2, We1, be1, We2, be2,
           Wl, Wr, Wedge, att, bias):
    src = edge_index[0]
    dst = edge_index[1]
    att_tab = jnp.tile(att[:, None], (1, 16))
    bias_rep = jnp.repeat(bias, N_PAD)

    t_tab = _node_encode(x, Wn1, bn1, Wn2, bn2, Wl, Wr)
    ee_list = _edge_encode(edge_attr.T, We1.T, be1, We2.T, Wedge.T, be2)
    nums, dens = _sc_attend(t_tab, src, dst, ee_list, att_tab)
    out_flat = _combine(nums, dens, bias_rep)
    return out_flat.reshape(OUT, N_PAD)[:, :N_NODES].T


# speedup vs baseline: 32.5288x; 1.9209x over previous
import functools

import jax
import jax.numpy as jnp
from jax import lax
from jax.experimental import pallas as pl
from jax.experimental.pallas import tpu as pltpu
from jax.experimental.pallas import tpu_sc as plsc

N_NODES = 10000
N_PAD = 10240
N_EDGES = 320000
OUT = 8

NUM_TILES = 32
EPT = N_EDGES // NUM_TILES
CHUNK = 400
NCHUNK = EPT // CHUNK
SUB = 80
NSUB = CHUNK // SUB
GROUPS = CHUNK // 16

_P = jax.lax.Precision.DEFAULT


def _dot(a, b):
    return jnp.dot(a, b, precision=_P, preferred_element_type=jnp.float32)


def _node_body(x_ref, wn1_ref, bn1_ref, wn2_ref, bn2_ref, wl_ref, wr_ref,
               t_ref):
    h1 = jnp.maximum(_dot(x_ref[...], wn1_ref[...]) + bn1_ref[...][None, :],
                     0.0)
    wlr = jnp.concatenate([wl_ref[...], wr_ref[...]], axis=1)
    w2 = _dot(wn2_ref[...], wlr)
    b2 = _dot(bn2_ref[...][None, :], wlr)
    t_ref[...] = _dot(h1, w2) + b2


def _node_encode(x, wn1, bn1, wn2, bn2, wl, wr):
    return pl.pallas_call(
        _node_body,
        out_shape=jax.ShapeDtypeStruct((N_NODES, 16), jnp.float32),
    )(x, wn1, bn1, wn2, bn2, wl, wr)


_EBLK = 6400
_EGRID = N_EDGES // _EBLK


def _edge_body(eat_ref, we1t_ref, be1_ref, we2t_ref, wedget_ref, be2_ref,
               *out_refs):
    i = pl.program_id(0)
    h = jnp.maximum(_dot(we1t_ref[...], eat_ref[...])
                    + be1_ref[...][:, None], 0.0)
    w2et = _dot(wedget_ref[...], we2t_ref[...])
    b2et = _dot(wedget_ref[...], be2_ref[...][:, None])
    eet = _dot(w2et, h) + b2et
    base = pl.multiple_of(i * _EBLK, 128)
    for f in range(OUT):
        out_refs[f][pl.ds(base, _EBLK)] = eet[f, :]


def _edge_encode(eat, we1t, be1, we2t, wedget, be2):
    return pl.pallas_call(
        _edge_body,
        grid=(_EGRID,),
        in_specs=[
            pl.BlockSpec((16, _EBLK), lambda i: (0, i)),
            pl.BlockSpec((128, 16), lambda i: (0, 0)),
            pl.BlockSpec((128,), lambda i: (0,)),
            pl.BlockSpec((128, 128), lambda i: (0, 0)),
            pl.BlockSpec((8, 128), lambda i: (0, 0)),
            pl.BlockSpec((128,), lambda i: (0,)),
        ],
        out_specs=[pl.BlockSpec((N_EDGES,), lambda i: (0,))
                   for _ in range(OUT)],
        out_shape=[jax.ShapeDtypeStruct((N_EDGES,), jnp.float32)
                   for _ in range(OUT)],
    )(eat, we1t, be1, we2t, wedget, be2)


def _sc_attend(t_tab, src, dst, ee_list, att_tab):
    mesh = plsc.VectorSubcoreMesh(core_axis_name="c", subcore_axis_name="s")

    @functools.partial(
        pl.kernel,
        out_type=(
            jax.ShapeDtypeStruct((NUM_TILES * N_PAD * OUT,), jnp.float32),
            jax.ShapeDtypeStruct((NUM_TILES * N_PAD,), jnp.float32),
        ),
        mesh=mesh,
        compiler_params=pltpu.CompilerParams(use_tc_tiling_on_sc=False,
                                             needs_layout_passes=False),
        scratch_types=[
            pltpu.VMEM((N_PAD * OUT,), jnp.float32),
            pltpu.VMEM((N_PAD,), jnp.float32),
            pltpu.VMEM((2, CHUNK, 16), jnp.float32),
            pltpu.VMEM((2, CHUNK, 16), jnp.float32),
            pltpu.VMEM((2, OUT, CHUNK), jnp.float32),
            pltpu.VMEM((3, CHUNK), jnp.int32),
            pltpu.VMEM((3, CHUNK), jnp.int32),
            pltpu.VMEM((OUT, 16), jnp.float32),
            pltpu.SemaphoreType.DMA,
            pltpu.SemaphoreType.DMA,
            pltpu.SemaphoreType.DMA,
        ],
    )
    def body(t_hbm, src_hbm, dst_hbm, e0, e1, e2, e3, e4, e5, e6, e7,
             att_hbm, nums_hbm, dens_hbm,
             num_v, den_v, srow_v, drow_v, ee_v, sidx_v, didx_v, att_v,
             sem_i, sem_r0, sem_r1):
        wid = lax.axis_index("s") * 2 + lax.axis_index("c")
        sem_r = [sem_r0, sem_r1]
        ee_hbms = [e0, e1, e2, e3, e4, e5, e6, e7]

        zero = jnp.zeros((16,), jnp.float32)

        def _znum(i, carry):
            base = i * 256
            for u in range(16):
                num_v[pl.ds(base + u * 16, 16)] = zero
            return carry

        def _zden(i, carry):
            base = i * 256
            for u in range(16):
                den_v[pl.ds(base + u * 16, 16)] = zero
            return carry

        lax.fori_loop(0, (N_PAD * OUT) // 256, _znum, 0)
        lax.fori_loop(0, N_PAD // 256, _zden, 0)

        pltpu.sync_copy(att_hbm, att_v)
        att_b = [att_v[f, :] for f in range(OUT)]

        def issue_idx(c):
            b3 = c % 3
            sl = pl.ds(wid * EPT + c * CHUNK, CHUNK)
            return [
                pltpu.async_copy(src_hbm.at[sl], sidx_v.at[b3], sem_i),
                pltpu.async_copy(dst_hbm.at[sl], didx_v.at[b3], sem_i),
            ]

        def issue_rows(c):
            b2, b3 = c % 2, c % 3
            hs = []
            for j in range(NSUB):
                sl = pl.ds(j * SUB, SUB)
                hs.append(pltpu.async_copy(
                    t_hbm.at[sidx_v.at[b3, sl]], srow_v.at[b2, sl],
                    sem_r[b2]))
                hs.append(pltpu.async_copy(
                    t_hbm.at[didx_v.at[b3, sl]], drow_v.at[b2, sl],
                    sem_r[b2]))
            esl = pl.ds(wid * EPT + c * CHUNK, CHUNK)
            for f in range(OUT):
                hs.append(pltpu.async_copy(
                    ee_hbms[f].at[esl], ee_v.at[b2, f], sem_r[b2]))
            return hs

        def compute_chunk(c):
            b2, b3 = c % 2, c % 3
            srow = srow_v.at[b2]
            drow = drow_v.at[b2]

            def gbody(g, carry):
                row = g * 16 + lax.iota(jnp.int32, 16)
                dstv = didx_v[b3, pl.ds(g * 16, 16)]
                acc = jnp.zeros((16,), jnp.float32)
                sls = []
                for f in range(OUT):
                    cf = jnp.full((16,), f, jnp.int32)
                    sl_f = plsc.load_gather(srow, [row, cf])
                    sr_f = plsc.load_gather(
                        drow, [row, jnp.full((16,), 8 + f, jnp.int32)])
                    e_f = ee_v[b2, f, pl.ds(g * 16, 16)]
                    m = sl_f + sr_f + e_f
                    lr = jnp.maximum(m, 0.2 * m)
                    acc = acc + att_b[f] * lr
                    sls.append(sl_f)
                ex = jnp.exp(acc)
                plsc.addupdate_scatter(den_v, [dstv], ex)
                for f in range(OUT):
                    plsc.addupdate_scatter(num_v, [dstv + f * N_PAD],
                                           ex * sls[f])
                return carry

            lax.fori_loop(0, GROUPS, gbody, 0)

        for h in issue_idx(0):
            h.wait()
        rows_pend = {0: issue_rows(0)}
        idx_pend = {1: issue_idx(1)}
        for c in range(NCHUNK):
            if c + 1 < NCHUNK:
                for h in idx_pend.pop(c + 1):
                    h.wait()
                rows_pend[c + 1] = issue_rows(c + 1)
            if c + 2 < NCHUNK:
                idx_pend[c + 2] = issue_idx(c + 2)
            for h in rows_pend.pop(c):
                h.wait()
            compute_chunk(c)

        pltpu.sync_copy(num_v, nums_hbm.at[pl.ds(wid * N_PAD * OUT,
                                                 N_PAD * OUT)])
        pltpu.sync_copy(den_v, dens_hbm.at[pl.ds(wid * N_PAD, N_PAD)])

    return body(t_tab, src, dst, *ee_list, att_tab)


def _combine_body(nums_ref, dens_ref, biasr_ref, out_ref, dacc_ref):
    i = pl.program_id(0)

    @pl.when(i == 0)
    def _init():
        out_ref[...] = jnp.zeros_like(out_ref)
        dacc_ref[...] = jnp.zeros_like(dacc_ref)

    out_ref[...] = out_ref[...] + nums_ref[...]
    dacc_ref[...] = dacc_ref[...] + dens_ref[...]

    @pl.when(i == NUM_TILES - 1)
    def _final():
        den_rep = jnp.concatenate([dacc_ref[...]] * OUT)
        out_ref[...] = out_ref[...] / (den_rep + 1e-16) + biasr_ref[...]


def _combine(nums, dens, bias_rep):
    return pl.pallas_call(
        _combine_body,
        grid=(NUM_TILES,),
        in_specs=[
            pl.BlockSpec((N_PAD * OUT,), lambda i: (i,)),
            pl.BlockSpec((N_PAD,), lambda i: (i,)),
            pl.BlockSpec((N_PAD * OUT,), lambda i: (0,)),
        ],
        out_specs=pl.BlockSpec((N_PAD * OUT,), lambda i: (0,)),
        out_shape=jax.ShapeDtypeStruct((N_PAD * OUT,), jnp.float32),
        scratch_shapes=[pltpu.VMEM((N_PAD,), jnp.float32)],
    )(nums, dens, bias_rep)


def kernel(x, edge_index, edge_attr, Wn1, bn1, Wn2, bn2, We1, be1, We2, be2,
           Wl, Wr, Wedge, att, bias):
    src = edge_index[0]
    dst = edge_index[1]
    att_tab = jnp.tile(att[:, None], (1, 16))
    bias_rep = jnp.repeat(bias, N_PAD)

    t_tab = _node_encode(x, Wn1, bn1, Wn2, bn2, Wl, Wr)
    ee_list = _edge_encode(edge_attr.T, We1.T, be1, We2.T, Wedge.T, be2)
    nums, dens = _sc_attend(t_tab, src, dst, ee_list, att_tab)
    out_flat = _combine(nums, dens, bias_rep)
    return out_flat.reshape(OUT, N_PAD)[:, :N_NODES].T


# speedup vs baseline: 34.6682x; 1.0658x over previous
import functools

import jax
import jax.numpy as jnp
from jax import lax
from jax.experimental import pallas as pl
from jax.experimental.pallas import tpu as pltpu
from jax.experimental.pallas import tpu_sc as plsc

N_NODES = 10000
N_PAD = 10240
N_EDGES = 320000
OUT = 8

NUM_TILES = 32
EPT = N_EDGES // NUM_TILES
CHUNK = 400
NCHUNK = EPT // CHUNK
SUB = 80
NSUB = CHUNK // SUB
GROUPS = CHUNK // 16

_P = jax.lax.Precision.DEFAULT


def _dot(a, b):
    return jnp.dot(a, b, precision=_P, preferred_element_type=jnp.float32)


def _node_body(x_ref, wn1_ref, bn1_ref, wn2_ref, bn2_ref, wl_ref, wr_ref,
               t_ref):
    h1 = jnp.maximum(_dot(x_ref[...], wn1_ref[...]) + bn1_ref[...][None, :],
                     0.0)
    wlr = jnp.concatenate([wl_ref[...], wr_ref[...]], axis=1)
    w2 = _dot(wn2_ref[...], wlr)
    b2 = _dot(bn2_ref[...][None, :], wlr)
    t_ref[...] = _dot(h1, w2) + b2


def _node_encode(x, wn1, bn1, wn2, bn2, wl, wr):
    return pl.pallas_call(
        _node_body,
        out_shape=jax.ShapeDtypeStruct((N_NODES, 16), jnp.float32),
    )(x, wn1, bn1, wn2, bn2, wl, wr)


_EBLK = 6400
_EGRID = N_EDGES // _EBLK


def _edge_body(eat_ref, we1t_ref, be1_ref, we2t_ref, wedget_ref, be2_ref,
               *out_refs):
    i = pl.program_id(0)
    h = jnp.maximum(_dot(we1t_ref[...], eat_ref[...])
                    + be1_ref[...][:, None], 0.0)
    w2et = _dot(wedget_ref[...], we2t_ref[...])
    b2et = _dot(wedget_ref[...], be2_ref[...][:, None])
    eet = _dot(w2et, h) + b2et
    base = pl.multiple_of(i * _EBLK, 128)
    for f in range(OUT):
        out_refs[f][pl.ds(base, _EBLK)] = eet[f, :]


def _edge_encode(eat, we1t, be1, we2t, wedget, be2):
    return pl.pallas_call(
        _edge_body,
        grid=(_EGRID,),
        in_specs=[
            pl.BlockSpec((16, _EBLK), lambda i: (0, i)),
            pl.BlockSpec((128, 16), lambda i: (0, 0)),
            pl.BlockSpec((128,), lambda i: (0,)),
            pl.BlockSpec((128, 128), lambda i: (0, 0)),
            pl.BlockSpec((8, 128), lambda i: (0, 0)),
            pl.BlockSpec((128,), lambda i: (0,)),
        ],
        out_specs=[pl.BlockSpec((N_EDGES,), lambda i: (0,))
                   for _ in range(OUT)],
        out_shape=[jax.ShapeDtypeStruct((N_EDGES,), jnp.float32)
                   for _ in range(OUT)],
    )(eat, we1t, be1, we2t, wedget, be2)


def _sc_attend(t_tab, src, dst, ee_list, att_tab):
    mesh = plsc.VectorSubcoreMesh(core_axis_name="c", subcore_axis_name="s")

    @functools.partial(
        pl.kernel,
        out_type=(
            jax.ShapeDtypeStruct((NUM_TILES * N_PAD * OUT,), jnp.float32),
            jax.ShapeDtypeStruct((NUM_TILES * N_PAD,), jnp.float32),
        ),
        mesh=mesh,
        compiler_params=pltpu.CompilerParams(use_tc_tiling_on_sc=False,
                                             needs_layout_passes=False),
        scratch_types=[
            pltpu.VMEM((N_PAD * OUT,), jnp.float32),
            pltpu.VMEM((N_PAD,), jnp.float32),
            pltpu.VMEM((2, CHUNK, 16), jnp.float32),
            pltpu.VMEM((2, CHUNK, 16), jnp.float32),
            pltpu.VMEM((2, OUT, CHUNK), jnp.float32),
            pltpu.VMEM((3, CHUNK), jnp.int32),
            pltpu.VMEM((3, CHUNK), jnp.int32),
            pltpu.VMEM((OUT, 16), jnp.float32),
            pltpu.SemaphoreType.DMA,
            pltpu.SemaphoreType.DMA,
            pltpu.SemaphoreType.DMA,
        ],
    )
    def body(t_hbm, src_hbm, dst_hbm, e0, e1, e2, e3, e4, e5, e6, e7,
             att_hbm, nums_hbm, dens_hbm,
             num_v, den_v, srow_v, drow_v, ee_v, sidx_v, didx_v, att_v,
             sem_i, sem_r0, sem_r1):
        wid = lax.axis_index("s") * 2 + lax.axis_index("c")
        sem_r = [sem_r0, sem_r1]
        ee_hbms = [e0, e1, e2, e3, e4, e5, e6, e7]

        def issue_idx(c):
            b3 = c % 3
            sl = pl.ds(wid * EPT + c * CHUNK, CHUNK)
            return [
                pltpu.async_copy(src_hbm.at[sl], sidx_v.at[b3], sem_i),
                pltpu.async_copy(dst_hbm.at[sl], didx_v.at[b3], sem_i),
            ]

        first_idx = issue_idx(0)

        zero = jnp.zeros((16,), jnp.float32)

        def _znum(i, carry):
            base = i * 256
            for u in range(16):
                num_v[pl.ds(base + u * 16, 16)] = zero
            return carry

        def _zden(i, carry):
            base = i * 256
            for u in range(16):
                den_v[pl.ds(base + u * 16, 16)] = zero
            return carry

        lax.fori_loop(0, (N_PAD * OUT) // 256, _znum, 0)
        lax.fori_loop(0, N_PAD // 256, _zden, 0)

        pltpu.sync_copy(att_hbm, att_v)
        att_b = [att_v[f, :] for f in range(OUT)]

        def issue_rows(c):
            b2, b3 = c % 2, c % 3
            hs = []
            for j in range(NSUB):
                sl = pl.ds(j * SUB, SUB)
                hs.append(pltpu.async_copy(
                    t_hbm.at[sidx_v.at[b3, sl]], srow_v.at[b2, sl],
                    sem_r[b2]))
                hs.append(pltpu.async_copy(
                    t_hbm.at[didx_v.at[b3, sl]], drow_v.at[b2, sl],
                    sem_r[b2]))
            esl = pl.ds(wid * EPT + c * CHUNK, CHUNK)
            for f in range(OUT):
                hs.append(pltpu.async_copy(
                    ee_hbms[f].at[esl], ee_v.at[b2, f], sem_r[b2]))
            return hs

        def compute_chunk(c):
            b2, b3 = c % 2, c % 3
            srow = srow_v.at[b2]
            drow = drow_v.at[b2]

            def gbody(g, carry):
                row = g * 16 + lax.iota(jnp.int32, 16)
                dstv = didx_v[b3, pl.ds(g * 16, 16)]
                acc = jnp.zeros((16,), jnp.float32)
                sls = []
                for f in range(OUT):
                    cf = jnp.full((16,), f, jnp.int32)
                    sl_f = plsc.load_gather(srow, [row, cf])
                    sr_f = plsc.load_gather(
                        drow, [row, jnp.full((16,), 8 + f, jnp.int32)])
                    e_f = ee_v[b2, f, pl.ds(g * 16, 16)]
                    m = sl_f + sr_f + e_f
                    lr = jnp.maximum(m, 0.2 * m)
                    acc = acc + att_b[f] * lr
                    sls.append(sl_f)
                ex = jnp.exp(acc)
                plsc.addupdate_scatter(den_v, [dstv], ex)
                for f in range(OUT):
                    plsc.addupdate_scatter(num_v, [dstv + f * N_PAD],
                                           ex * sls[f])
                return carry

            lax.fori_loop(0, GROUPS, gbody, 0)

        for h in first_idx:
            h.wait()
        rows_pend = {0: issue_rows(0)}
        idx_pend = {1: issue_idx(1)}
        for c in range(NCHUNK):
            if c + 1 < NCHUNK:
                for h in idx_pend.pop(c + 1):
                    h.wait()
                rows_pend[c + 1] = issue_rows(c + 1)
            if c + 2 < NCHUNK:
                idx_pend[c + 2] = issue_idx(c + 2)
            for h in rows_pend.pop(c):
                h.wait()
            compute_chunk(c)

        pltpu.sync_copy(num_v, nums_hbm.at[pl.ds(wid * N_PAD * OUT,
                                                 N_PAD * OUT)])
        pltpu.sync_copy(den_v, dens_hbm.at[pl.ds(wid * N_PAD, N_PAD)])

    return body(t_tab, src, dst, *ee_list, att_tab)


_CB = 4
_CGRID = NUM_TILES // _CB


def _combine_body(nums_ref, dens_ref, biasr_ref, out_ref, dacc_ref):
    i = pl.program_id(0)

    @pl.when(i == 0)
    def _init():
        out_ref[...] = jnp.zeros_like(out_ref)
        dacc_ref[...] = jnp.zeros_like(dacc_ref)

    w = N_PAD * OUT
    nsum = nums_ref[pl.ds(0, w)]
    dsum = dens_ref[pl.ds(0, N_PAD)]
    for t in range(1, _CB):
        nsum = nsum + nums_ref[pl.ds(t * w, w)]
        dsum = dsum + dens_ref[pl.ds(t * N_PAD, N_PAD)]
    out_ref[...] = out_ref[...] + nsum
    dacc_ref[...] = dacc_ref[...] + dsum

    @pl.when(i == _CGRID - 1)
    def _final():
        den_rep = jnp.concatenate([dacc_ref[...]] * OUT)
        out_ref[...] = out_ref[...] / (den_rep + 1e-16) + biasr_ref[...]


def _combine(nums, dens, bias_rep):
    return pl.pallas_call(
        _combine_body,
        grid=(_CGRID,),
        in_specs=[
            pl.BlockSpec((_CB * N_PAD * OUT,), lambda i: (i,)),
            pl.BlockSpec((_CB * N_PAD,), lambda i: (i,)),
            pl.BlockSpec((N_PAD * OUT,), lambda i: (0,)),
        ],
        out_specs=pl.BlockSpec((N_PAD * OUT,), lambda i: (0,)),
        out_shape=jax.ShapeDtypeStruct((N_PAD * OUT,), jnp.float32),
        scratch_shapes=[pltpu.VMEM((N_PAD,), jnp.float32)],
    )(nums, dens, bias_rep)


def kernel(x, edge_index, edge_attr, Wn1, bn1, Wn2, bn2, We1, be1, We2, be2,
           Wl, Wr, Wedge, att, bias):
    src = edge_index[0]
    dst = edge_index[1]
    att_tab = jnp.tile(att[:, None], (1, 16))
    bias_rep = jnp.repeat(bias, N_PAD)

    t_tab = _node_encode(x, Wn1, bn1, Wn2, bn2, Wl, Wr)
    ee_list = _edge_encode(edge_attr.T, We1.T, be1, We2.T, Wedge.T, be2)
    nums, dens = _sc_attend(t_tab, src, dst, ee_list, att_tab)
    out_flat = _combine(nums, dens, bias_rep)
    return out_flat.reshape(OUT, N_PAD)[:, :N_NODES].T
